# baseline (speedup 1.0000x reference)
import jax
import jax.numpy as jnp
from jax.experimental import pallas as pl


def _mm_kernel(a_ref, w_ref, o_ref):
    w = jnp.maximum(w_ref[...], 0.0).astype(jnp.bfloat16)
    a = a_ref[...].astype(jnp.bfloat16)
    o_ref[...] = jnp.dot(a, w, preferred_element_type=jnp.float32)


def kernel(user_ids, item_ids, explicit_feedback, dense_weight_slice):
    M, K = explicit_feedback.shape
    N = dense_weight_slice.shape[1]
    bm = 256
    return pl.pallas_call(
        _mm_kernel,
        grid=(M // bm,),
        in_specs=[
            pl.BlockSpec((bm, K), lambda i: (i, 0)),
            pl.BlockSpec((K, N), lambda i: (0, 0)),
        ],
        out_specs=pl.BlockSpec((bm, N), lambda i: (i, 0)),
        out_shape=jax.ShapeDtypeStruct((M, N), jnp.float32),
    )(explicit_feedback, dense_weight_slice)


# speedup vs baseline: 1.0924x; 1.0924x over previous
import jax
import jax.numpy as jnp
from jax.experimental import pallas as pl


def _mm_kernel(a_ref, w_ref, o_ref):
    w = jnp.maximum(w_ref[...], 0.0).astype(jnp.bfloat16)
    a = a_ref[...].astype(jnp.bfloat16)
    o_ref[...] = jnp.dot(a, w, preferred_element_type=jnp.float32)


def kernel(user_ids, item_ids, explicit_feedback, dense_weight_slice):
    M, K = explicit_feedback.shape
    N = dense_weight_slice.shape[1]
    bm = 1024
    return pl.pallas_call(
        _mm_kernel,
        grid=(M // bm,),
        in_specs=[
            pl.BlockSpec((bm, K), lambda i: (i, 0)),
            pl.BlockSpec((K, N), lambda i: (0, 0)),
        ],
        out_specs=pl.BlockSpec((bm, N), lambda i: (i, 0)),
        out_shape=jax.ShapeDtypeStruct((M, N), jnp.float32),
    )(explicit_feedback, dense_weight_slice)


# speedup vs baseline: 1.1548x; 1.0572x over previous
import jax
import jax.numpy as jnp
from jax.experimental import pallas as pl

_NSPLIT = 4


def _mm_kernel(a0_ref, a1_ref, a2_ref, a3_ref, w_ref, o_ref):
    w = jnp.maximum(w_ref[...], 0.0).astype(jnp.bfloat16)
    kc = w.shape[0] // _NSPLIT
    acc = None
    for j, a_ref in enumerate((a0_ref, a1_ref, a2_ref, a3_ref)):
        a = a_ref[...].astype(jnp.bfloat16)
        p = jnp.dot(a, w[j * kc:(j + 1) * kc, :],
                    preferred_element_type=jnp.float32)
        acc = p if acc is None else acc + p
    o_ref[...] = acc


def kernel(user_ids, item_ids, explicit_feedback, dense_weight_slice):
    M, K = explicit_feedback.shape
    N = dense_weight_slice.shape[1]
    bm = 512
    kc = K // _NSPLIT
    a_specs = [
        pl.BlockSpec((bm, kc), lambda i, j=j: (i, j)) for j in range(_NSPLIT)
    ]
    return pl.pallas_call(
        _mm_kernel,
        grid=(M // bm,),
        in_specs=a_specs + [pl.BlockSpec((K, N), lambda i: (0, 0))],
        out_specs=pl.BlockSpec((bm, N), lambda i: (i, 0)),
        out_shape=jax.ShapeDtypeStruct((M, N), jnp.float32),
    )(*([explicit_feedback] * _NSPLIT), dense_weight_slice)
